# initial kernel scaffold (unmeasured)
import jax
import jax.numpy as jnp
from jax import lax
from jax.experimental import pallas as pl
from jax.experimental.pallas import tpu as pltpu


def kernel(
    x,
):
    def body(*refs):
        pass

    out_shape = jax.ShapeDtypeStruct(..., jnp.float32)
    return pl.pallas_call(body, out_shape=out_shape)(...)


# baseline (device time: 17716 ns/iter reference)
import jax
import jax.numpy as jnp
from jax import lax
from jax.experimental import pallas as pl
from jax.experimental.pallas import tpu as pltpu


def kernel(x):
    m, n = x.shape

    def body(x_ref, out_ref, xb_ref, comm_ref, send_sem, recv_sem):
        my_x = lax.axis_index("x")
        my_y = lax.axis_index("y")
        nbr = (1 - my_x, my_y)

        xb_ref[...] = x_ref[...].astype(jnp.bfloat16)

        barrier = pltpu.get_barrier_semaphore()
        pl.semaphore_signal(
            barrier, inc=1, device_id=nbr, device_id_type=pl.DeviceIdType.MESH
        )
        pl.semaphore_wait(barrier, 1)

        rdma = pltpu.make_async_remote_copy(
            src_ref=xb_ref,
            dst_ref=comm_ref,
            send_sem=send_sem,
            recv_sem=recv_sem,
            device_id=nbr,
            device_id_type=pl.DeviceIdType.MESH,
        )
        rdma.start()
        rdma.wait()

        out_ref[...] = (
            x_ref[...] + comm_ref[...].astype(jnp.float32)
        ).astype(jnp.bfloat16)

    return pl.pallas_call(
        body,
        out_shape=jax.ShapeDtypeStruct((m, n), jnp.bfloat16),
        in_specs=[pl.BlockSpec(memory_space=pltpu.VMEM)],
        out_specs=pl.BlockSpec(memory_space=pltpu.VMEM),
        scratch_shapes=[
            pltpu.VMEM((m, n), jnp.bfloat16),
            pltpu.VMEM((m, n), jnp.bfloat16),
            pltpu.SemaphoreType.DMA,
            pltpu.SemaphoreType.DMA,
        ],
        compiler_params=pltpu.CompilerParams(collective_id=0),
    )(x)


# device time: 15266 ns/iter; 1.1605x vs baseline; 1.1605x over previous
import jax
import jax.numpy as jnp
from jax import lax
from jax.experimental import pallas as pl
from jax.experimental.pallas import tpu as pltpu

C = 8


def kernel(x):
    m, n = x.shape
    h = m // 2
    ch = h // C

    def body(x_ref, out_ref, sx_ref, rx_ref, red_ref, ry_ref,
             sx_sems, rx_sems, sy_sems, ry_sems):
        my_x = lax.axis_index("x")
        my_y = lax.axis_index("y")
        xn = (1 - my_x, my_y)
        yn = (my_x, 1 - my_y)
        base = my_y * h

        barrier = pltpu.get_barrier_semaphore()
        for nbr in (xn, yn):
            pl.semaphore_signal(
                barrier, inc=1, device_id=nbr,
                device_id_type=pl.DeviceIdType.MESH,
            )
        pl.semaphore_wait(barrier, 2)

        rdma_x = []
        rdma_y = []
        for c in range(C):
            s = pl.ds(c * ch, ch)
            rdma_x.append(pltpu.make_async_remote_copy(
                src_ref=sx_ref.at[s],
                dst_ref=rx_ref.at[s],
                send_sem=sx_sems.at[c],
                recv_sem=rx_sems.at[c],
                device_id=xn,
                device_id_type=pl.DeviceIdType.MESH,
            ))
            rdma_y.append(pltpu.make_async_remote_copy(
                src_ref=red_ref.at[s],
                dst_ref=ry_ref.at[s],
                send_sem=sy_sems.at[c],
                recv_sem=ry_sems.at[c],
                device_id=yn,
                device_id_type=pl.DeviceIdType.MESH,
            ))

        for c in range(C):
            s = pl.ds(c * ch, ch)
            sx_ref[s, :] = x_ref[pl.ds(base + c * ch, ch), :].astype(
                jnp.bfloat16
            )
            rdma_x[c].start()

        for c in range(C):
            s = pl.ds(c * ch, ch)
            rdma_x[c].wait_recv()
            red_ref[s, :] = sx_ref[s, :] + rx_ref[s, :]
            rdma_y[c].start()

        out_ref[pl.ds(base, h), :] = red_ref[...]

        for c in range(C):
            rdma_y[c].wait_recv()
        out_ref[pl.ds((1 - my_y) * h, h), :] = ry_ref[...]

        for c in range(C):
            rdma_x[c].wait_send()
            rdma_y[c].wait_send()

    return pl.pallas_call(
        body,
        out_shape=jax.ShapeDtypeStruct((m, n), jnp.bfloat16),
        in_specs=[pl.BlockSpec(memory_space=pltpu.VMEM)],
        out_specs=pl.BlockSpec(memory_space=pltpu.VMEM),
        scratch_shapes=[
            pltpu.VMEM((h, n), jnp.bfloat16),
            pltpu.VMEM((h, n), jnp.bfloat16),
            pltpu.VMEM((h, n), jnp.bfloat16),
            pltpu.VMEM((h, n), jnp.bfloat16),
            pltpu.SemaphoreType.DMA((C,)),
            pltpu.SemaphoreType.DMA((C,)),
            pltpu.SemaphoreType.DMA((C,)),
            pltpu.SemaphoreType.DMA((C,)),
        ],
        compiler_params=pltpu.CompilerParams(collective_id=0),
    )(x)
